# baseline (speedup 1.0000x reference)
import functools

import jax
import jax.numpy as jnp
from jax import lax
from jax.experimental import pallas as pl
from jax.experimental.pallas import tpu as pltpu
from jax.experimental.pallas import tpu_sc as plsc

_NC, _NS = 2, 16
_NW = _NC * _NS


def _router_body(x_ref, wr_ref, xs_ref, dest_ref, off_ref):
    S, H = x_ref.shape
    E = wr_ref.shape[1]
    C = 128
    NCH = S // C

    x = x_ref[...]
    logits = jnp.dot(x.astype(jnp.bfloat16), wr_ref[...].astype(jnp.bfloat16),
                     preferred_element_type=jnp.float32)
    top = jnp.max(logits, axis=1)
    iota_e = lax.broadcasted_iota(jnp.int32, (S, E), 1)
    e_idx = jnp.min(jnp.where(logits == top[:, None], iota_e, E), axis=1)
    score = 1.0 / (1.0 + jnp.exp(-top))

    onehot = (iota_e == e_idx[:, None]).astype(jnp.float32)

    tri = (lax.broadcasted_iota(jnp.int32, (C, C), 0)
           > lax.broadcasted_iota(jnp.int32, (C, C), 1)).astype(jnp.bfloat16)
    base = jnp.zeros((1, E), jnp.float32)
    rank_chunks = []
    for c in range(NCH):
        chunk = lax.slice(onehot, (c * C, 0), ((c + 1) * C, E))
        pre = jnp.dot(tri, chunk.astype(jnp.bfloat16),
                      preferred_element_type=jnp.float32)
        rank_chunks.append(jnp.sum((pre + base) * chunk, axis=1))
        base = base + jnp.sum(chunk, axis=0, keepdims=True)
    rank = jnp.concatenate(rank_chunks, axis=0)
    counts = base

    ut = (lax.broadcasted_iota(jnp.int32, (E, E), 0)
          < lax.broadcasted_iota(jnp.int32, (E, E), 1)).astype(jnp.bfloat16)
    c_hi = jnp.floor(counts / 256.0) * 256.0
    c_lo = counts - c_hi
    off = (jnp.dot(c_hi.astype(jnp.bfloat16), ut,
                   preferred_element_type=jnp.float32)
           + jnp.dot(c_lo.astype(jnp.bfloat16), ut,
                     preferred_element_type=jnp.float32))

    off_tok = jnp.sum(onehot * off, axis=1)
    dest_ref[...] = (off_tok + rank).astype(jnp.int32)

    off_row = jnp.concatenate(
        [off.reshape(E), jnp.full((E,), S, jnp.float32)])
    off_ref[...] = off_row.astype(jnp.int32)

    xs_ref[...] = x * score[:, None]


def _router(x, wr):
    S, H = x.shape
    E = wr.shape[1]
    return pl.pallas_call(
        _router_body,
        out_shape=(
            jax.ShapeDtypeStruct((S, H), jnp.float32),
            jax.ShapeDtypeStruct((S,), jnp.int32),
            jax.ShapeDtypeStruct((2 * E,), jnp.int32),
        ),
    )(x, wr)


def _sc_scatter(xs, dest):
    S, H = xs.shape
    rows_per_w = S // _NW
    mesh = plsc.VectorSubcoreMesh(core_axis_name="c", subcore_axis_name="s")

    @functools.partial(
        pl.kernel, mesh=mesh,
        out_type=jax.ShapeDtypeStruct((S, H), jnp.float32),
        scratch_types=[
            pltpu.VMEM((rows_per_w,), jnp.int32),
            pltpu.VMEM((rows_per_w, H), jnp.float32),
            pltpu.SemaphoreType.DMA,
        ],
    )
    def k(xs_hbm, dest_hbm, out_hbm, idx_v, rows_v, sem):
        wid = lax.axis_index("s") * _NC + lax.axis_index("c")
        base = wid * rows_per_w
        pltpu.sync_copy(dest_hbm.at[pl.ds(base, rows_per_w)], idx_v)
        pltpu.sync_copy(xs_hbm.at[pl.ds(base, rows_per_w)], rows_v)
        pltpu.async_copy(rows_v, out_hbm.at[idx_v], sem).wait()

    return k(xs, dest)


def _sc_gather(y_sorted, dest):
    S, H = y_sorted.shape
    rows_per_w = S // _NW
    mesh = plsc.VectorSubcoreMesh(core_axis_name="c", subcore_axis_name="s")

    @functools.partial(
        pl.kernel, mesh=mesh,
        out_type=jax.ShapeDtypeStruct((S, H), jnp.float32),
        scratch_types=[
            pltpu.VMEM((rows_per_w,), jnp.int32),
            pltpu.VMEM((rows_per_w, H), jnp.float32),
            pltpu.SemaphoreType.DMA,
        ],
    )
    def k(y_hbm, dest_hbm, out_hbm, idx_v, rows_v, sem):
        wid = lax.axis_index("s") * _NC + lax.axis_index("c")
        base = wid * rows_per_w
        pltpu.sync_copy(dest_hbm.at[pl.ds(base, rows_per_w)], idx_v)
        pltpu.async_copy(y_hbm.at[idx_v], rows_v, sem).wait()
        pltpu.sync_copy(rows_v, out_hbm.at[pl.ds(base, rows_per_w)])

    return k(y_sorted, dest)


_T = 256


def _gmm_body(off_ref, x_ref, wg_ref, wu_ref, wd_ref, y_ref):
    e = pl.program_id(0)
    f = pl.program_id(1)
    S, H = x_ref.shape

    @pl.when((e == 0) & (f == 0))
    def _():
        y_ref[...] = jnp.zeros_like(y_ref)

    start = off_ref[e]
    end = off_ref[e + 1]
    astart = (start // 8) * 8
    nchunks = jnp.where(end > start, (end - astart + _T - 1) // _T, 0)

    def chunk(k, carry):
        a = astart + k * _T
        s = pl.multiple_of(jnp.minimum(a, S - _T), 8)
        gid = s + lax.broadcasted_iota(jnp.int32, (_T, 1), 0)
        m = ((gid >= jnp.maximum(start, a))
             & (gid < jnp.minimum(a + _T, end))).astype(jnp.float32)
        xc = (x_ref[pl.ds(s, _T), :] * m).astype(jnp.bfloat16)
        g = jnp.dot(xc, wg_ref[0], preferred_element_type=jnp.float32)
        u = jnp.dot(xc, wu_ref[0], preferred_element_type=jnp.float32)
        h = (g * (1.0 / (1.0 + jnp.exp(-g)))) * u
        y_ref[pl.ds(s, _T), :] += jnp.dot(
            h.astype(jnp.bfloat16), wd_ref[0],
            preferred_element_type=jnp.float32)
        return carry

    lax.fori_loop(0, nchunks, chunk, 0)


def _gmm(off, x_sorted, Wg, Wu, Wd):
    S, H = x_sorted.shape
    E, _, F = Wg.shape
    F2 = F // 2
    grid_spec = pltpu.PrefetchScalarGridSpec(
        num_scalar_prefetch=1,
        grid=(E, 2),
        in_specs=[
            pl.BlockSpec((S, H), lambda e, f, off: (0, 0)),
            pl.BlockSpec((1, H, F2), lambda e, f, off: (e, 0, f)),
            pl.BlockSpec((1, H, F2), lambda e, f, off: (e, 0, f)),
            pl.BlockSpec((1, F2, H), lambda e, f, off: (e, f, 0)),
        ],
        out_specs=pl.BlockSpec((S, H), lambda e, f, off: (0, 0)),
    )
    return pl.pallas_call(
        _gmm_body,
        grid_spec=grid_spec,
        out_shape=jax.ShapeDtypeStruct((S, H), jnp.float32),
        compiler_params=pltpu.CompilerParams(
            dimension_semantics=("arbitrary", "arbitrary")),
    )(off, x_sorted, Wg, Wu, Wd)


def _shared_body(x_ref, wg_ref, wu_ref, wd_ref, o_ref):
    f = pl.program_id(0)
    S, H = x_ref.shape
    for r in range(S // _T):
        sl = pl.ds(r * _T, _T)
        xc = x_ref[sl, :].astype(jnp.bfloat16)
        g = jnp.dot(xc, wg_ref[...], preferred_element_type=jnp.float32)
        u = jnp.dot(xc, wu_ref[...], preferred_element_type=jnp.float32)
        h = (g * (1.0 / (1.0 + jnp.exp(-g)))) * u
        part = jnp.dot(h.astype(jnp.bfloat16), wd_ref[...],
                       preferred_element_type=jnp.float32)

        @pl.when(f == 0)
        def _():
            o_ref[sl, :] = part

        @pl.when(f != 0)
        def _():
            o_ref[sl, :] += part


def _shared(x, sWg, sWu, sWd):
    S, H = x.shape
    F = sWg.shape[1]
    F2 = F // 2
    return pl.pallas_call(
        _shared_body,
        grid=(2,),
        in_specs=[
            pl.BlockSpec((S, H), lambda f: (0, 0)),
            pl.BlockSpec((H, F2), lambda f: (0, f)),
            pl.BlockSpec((H, F2), lambda f: (0, f)),
            pl.BlockSpec((F2, H), lambda f: (f, 0)),
        ],
        out_specs=pl.BlockSpec((S, H), lambda f: (0, 0)),
        out_shape=jax.ShapeDtypeStruct((S, H), jnp.float32),
        compiler_params=pltpu.CompilerParams(
            dimension_semantics=("arbitrary",)),
    )(x, sWg, sWu, sWd)


def _add_body(a_ref, b_ref, o_ref):
    o_ref[...] = a_ref[...] + b_ref[...]


def _add(a, b):
    return pl.pallas_call(
        _add_body,
        out_shape=jax.ShapeDtypeStruct(a.shape, a.dtype),
    )(a, b)


def kernel(hidden_states, W_router, Wg, Wu, Wd, sWg, sWu, sWd):
    S, B, H = hidden_states.shape
    x = hidden_states.reshape(S * B, H)
    xs, dest, off = _router(x, W_router)
    x_sorted = _sc_scatter(xs, dest)
    sh = _shared(x, sWg, sWu, sWd)
    y_sorted = _gmm(off, x_sorted, Wg, Wu, Wd)
    y_tok = _sc_gather(y_sorted, dest)
    out = _add(sh, y_tok)
    return out.reshape(S, B, H)


# speedup vs baseline: 1.0378x; 1.0378x over previous
import functools

import jax
import jax.numpy as jnp
from jax import lax
from jax.experimental import pallas as pl
from jax.experimental.pallas import tpu as pltpu
from jax.experimental.pallas import tpu_sc as plsc

_NC, _NS = 2, 16
_NW = _NC * _NS


def _router_body(x_ref, wr_ref, xs_ref, dest_ref, off_ref):
    S, H = x_ref.shape
    E = wr_ref.shape[1]
    C = 128
    NCH = S // C

    x = x_ref[...]
    logits = jnp.dot(x.astype(jnp.bfloat16), wr_ref[...].astype(jnp.bfloat16),
                     preferred_element_type=jnp.float32)
    top = jnp.max(logits, axis=1)
    iota_e = lax.broadcasted_iota(jnp.int32, (S, E), 1)
    e_idx = jnp.min(jnp.where(logits == top[:, None], iota_e, E), axis=1)
    score = 1.0 / (1.0 + jnp.exp(-top))

    onehot = (iota_e == e_idx[:, None]).astype(jnp.float32)

    tri = (lax.broadcasted_iota(jnp.int32, (C, C), 0)
           > lax.broadcasted_iota(jnp.int32, (C, C), 1)).astype(jnp.bfloat16)
    base = jnp.zeros((1, E), jnp.float32)
    rank_chunks = []
    for c in range(NCH):
        chunk = lax.slice(onehot, (c * C, 0), ((c + 1) * C, E))
        pre = jnp.dot(tri, chunk.astype(jnp.bfloat16),
                      preferred_element_type=jnp.float32)
        rank_chunks.append(jnp.sum((pre + base) * chunk, axis=1))
        base = base + jnp.sum(chunk, axis=0, keepdims=True)
    rank = jnp.concatenate(rank_chunks, axis=0)
    counts = base

    ut = (lax.broadcasted_iota(jnp.int32, (E, E), 0)
          < lax.broadcasted_iota(jnp.int32, (E, E), 1)).astype(jnp.bfloat16)
    c_hi = jnp.floor(counts / 256.0) * 256.0
    c_lo = counts - c_hi
    off = (jnp.dot(c_hi.astype(jnp.bfloat16), ut,
                   preferred_element_type=jnp.float32)
           + jnp.dot(c_lo.astype(jnp.bfloat16), ut,
                     preferred_element_type=jnp.float32))

    off_tok = jnp.sum(onehot * off, axis=1)
    dest_ref[...] = (off_tok + rank).astype(jnp.int32)

    off_row = jnp.concatenate(
        [off.reshape(E), jnp.full((E,), S, jnp.float32)])
    off_ref[...] = off_row.astype(jnp.int32)

    xs = x * score[:, None]
    NP = H // 128
    for c in range(NP):
        xs_ref[c] = lax.slice(xs, (0, 128 * c), (S, 128 * (c + 1)))


def _router(x, wr):
    S, H = x.shape
    E = wr.shape[1]
    return pl.pallas_call(
        _router_body,
        out_shape=(
            jax.ShapeDtypeStruct((H // 128, S, 128), jnp.float32),
            jax.ShapeDtypeStruct((S,), jnp.int32),
            jax.ShapeDtypeStruct((2 * E,), jnp.int32),
        ),
    )(x, wr)


def _sc_scatter(xs, dest):
    NP, S, L = xs.shape
    rows_per_w = S // _NW
    mesh = plsc.VectorSubcoreMesh(core_axis_name="c", subcore_axis_name="s")

    @functools.partial(
        pl.kernel, mesh=mesh,
        out_type=jax.ShapeDtypeStruct((NP, S, L), jnp.float32),
        scratch_types=[
            pltpu.VMEM((rows_per_w,), jnp.int32),
            pltpu.VMEM((NP, rows_per_w, L), jnp.float32),
            pltpu.SemaphoreType.DMA,
        ],
    )
    def k(xs_hbm, dest_hbm, out_hbm, idx_v, rows_v, sem):
        wid = lax.axis_index("s") * _NC + lax.axis_index("c")
        base = wid * rows_per_w
        pltpu.sync_copy(dest_hbm.at[pl.ds(base, rows_per_w)], idx_v)
        pltpu.sync_copy(xs_hbm.at[:, pl.ds(base, rows_per_w)], rows_v)
        cps = [pltpu.async_copy(rows_v.at[c], out_hbm.at[c].at[idx_v], sem)
               for c in range(NP)]
        for cp in cps:
            cp.wait()

    return k(xs, dest)


def _sc_gather(y_sorted, dest):
    NP, S, L = y_sorted.shape
    rows_per_w = S // _NW
    mesh = plsc.VectorSubcoreMesh(core_axis_name="c", subcore_axis_name="s")

    @functools.partial(
        pl.kernel, mesh=mesh,
        out_type=jax.ShapeDtypeStruct((NP, S, L), jnp.float32),
        scratch_types=[
            pltpu.VMEM((rows_per_w,), jnp.int32),
            pltpu.VMEM((NP, rows_per_w, L), jnp.float32),
            pltpu.SemaphoreType.DMA,
        ],
    )
    def k(y_hbm, dest_hbm, out_hbm, idx_v, rows_v, sem):
        wid = lax.axis_index("s") * _NC + lax.axis_index("c")
        base = wid * rows_per_w
        pltpu.sync_copy(dest_hbm.at[pl.ds(base, rows_per_w)], idx_v)
        cps = [pltpu.async_copy(y_hbm.at[c].at[idx_v], rows_v.at[c], sem)
               for c in range(NP)]
        for cp in cps:
            cp.wait()
        pltpu.sync_copy(rows_v, out_hbm.at[:, pl.ds(base, rows_per_w)])

    return k(y_sorted, dest)


_T = 256


def _gmm_body(off_ref, x_ref, wg_ref, wu_ref, wd_ref, y_ref):
    e = pl.program_id(0)
    f = pl.program_id(1)
    NP, S, L = x_ref.shape

    @pl.when((e == 0) & (f == 0))
    def _():
        y_ref[...] = jnp.zeros_like(y_ref)

    start = off_ref[e]
    end = off_ref[e + 1]
    astart = (start // 8) * 8
    nchunks = jnp.where(end > start, (end - astart + _T - 1) // _T, 0)

    def chunk(k, carry):
        a = astart + k * _T
        s = pl.multiple_of(jnp.minimum(a, S - _T), 8)
        gid = s + lax.broadcasted_iota(jnp.int32, (_T, 1), 0)
        m = ((gid >= jnp.maximum(start, a))
             & (gid < jnp.minimum(a + _T, end))).astype(jnp.float32)
        xc = jnp.concatenate(
            [x_ref[c, pl.ds(s, _T), :] for c in range(NP)], axis=1)
        xc = (xc * m).astype(jnp.bfloat16)
        g = jnp.dot(xc, wg_ref[0], preferred_element_type=jnp.float32)
        u = jnp.dot(xc, wu_ref[0], preferred_element_type=jnp.float32)
        h = (g * (1.0 / (1.0 + jnp.exp(-g)))) * u
        part = jnp.dot(h.astype(jnp.bfloat16), wd_ref[0],
                       preferred_element_type=jnp.float32)
        for c in range(NP):
            y_ref[c, pl.ds(s, _T), :] += lax.slice(
                part, (0, L * c), (_T, L * (c + 1)))
        return carry

    lax.fori_loop(0, nchunks, chunk, 0)


def _gmm(off, x_sorted, Wg, Wu, Wd):
    NP, S, L = x_sorted.shape
    E, H, F = Wg.shape
    F2 = F // 2
    grid_spec = pltpu.PrefetchScalarGridSpec(
        num_scalar_prefetch=1,
        grid=(E, 2),
        in_specs=[
            pl.BlockSpec((NP, S, L), lambda e, f, off: (0, 0, 0)),
            pl.BlockSpec((1, H, F2), lambda e, f, off: (e, 0, f)),
            pl.BlockSpec((1, H, F2), lambda e, f, off: (e, 0, f)),
            pl.BlockSpec((1, F2, H), lambda e, f, off: (e, f, 0)),
        ],
        out_specs=pl.BlockSpec((NP, S, L), lambda e, f, off: (0, 0, 0)),
    )
    return pl.pallas_call(
        _gmm_body,
        grid_spec=grid_spec,
        out_shape=jax.ShapeDtypeStruct((NP, S, L), jnp.float32),
        compiler_params=pltpu.CompilerParams(
            dimension_semantics=("arbitrary", "arbitrary")),
    )(off, x_sorted, Wg, Wu, Wd)


def _shared_body(x_ref, wg_ref, wu_ref, wd_ref, yt_ref, o_ref):
    f = pl.program_id(0)
    S, H = x_ref.shape
    NP, _, L = yt_ref.shape
    for r in range(S // _T):
        sl = pl.ds(r * _T, _T)
        xc = x_ref[sl, :].astype(jnp.bfloat16)
        g = jnp.dot(xc, wg_ref[...], preferred_element_type=jnp.float32)
        u = jnp.dot(xc, wu_ref[...], preferred_element_type=jnp.float32)
        h = (g * (1.0 / (1.0 + jnp.exp(-g)))) * u
        part = jnp.dot(h.astype(jnp.bfloat16), wd_ref[...],
                       preferred_element_type=jnp.float32)

        @pl.when(f == 0)
        def _():
            yt = jnp.concatenate(
                [yt_ref[c, sl, :] for c in range(NP)], axis=1)
            o_ref[sl, :] = yt + part

        @pl.when(f != 0)
        def _():
            o_ref[sl, :] += part


def _shared(x, sWg, sWu, sWd, y_tok):
    S, H = x.shape
    F = sWg.shape[1]
    F2 = F // 2
    NP, _, L = y_tok.shape
    return pl.pallas_call(
        _shared_body,
        grid=(2,),
        in_specs=[
            pl.BlockSpec((S, H), lambda f: (0, 0)),
            pl.BlockSpec((H, F2), lambda f: (0, f)),
            pl.BlockSpec((H, F2), lambda f: (0, f)),
            pl.BlockSpec((F2, H), lambda f: (f, 0)),
            pl.BlockSpec((NP, S, L), lambda f: (0, 0, 0)),
        ],
        out_specs=pl.BlockSpec((S, H), lambda f: (0, 0)),
        out_shape=jax.ShapeDtypeStruct((S, H), jnp.float32),
        compiler_params=pltpu.CompilerParams(
            dimension_semantics=("arbitrary",)),
    )(x, sWg, sWu, sWd, y_tok)


def kernel(hidden_states, W_router, Wg, Wu, Wd, sWg, sWu, sWd):
    S, B, H = hidden_states.shape
    x = hidden_states.reshape(S * B, H)
    xs, dest, off = _router(x, W_router)
    x_sorted = _sc_scatter(xs, dest)
    y_sorted = _gmm(off, x_sorted, Wg, Wu, Wd)
    y_tok = _sc_gather(y_sorted, dest)
    out = _shared(x, sWg, sWu, sWd, y_tok)
    return out.reshape(S, B, H)
